# Initial kernel scaffold; baseline (speedup 1.0000x reference)
import jax
import jax.numpy as jnp
from jax.experimental import pallas as pl


def kernel(x, table):
    raise NotImplementedError("write your pallas kernel here")


# speedup vs baseline: 1.6850x; 1.6850x over previous
import functools

import jax
import jax.numpy as jnp
from jax import lax
from jax.experimental import pallas as pl
from jax.experimental.pallas import tpu as pltpu
from jax.experimental.pallas import tpu_sc as plsc

VOCAB = 1000000
D = 64
B = 16384 * 50
NC = 2
NS = 16
NW = NC * NS
BPW = B // NW
CHUNK = 128
NCH = BPW // CHUNK

_mesh = plsc.VectorSubcoreMesh(core_axis_name="c", subcore_axis_name="s")


@functools.partial(
    pl.kernel,
    out_type=jax.ShapeDtypeStruct((B, D), jnp.float32),
    mesh=_mesh,
    scratch_types=[
        pltpu.VMEM((NCH, CHUNK), jnp.int32),
        pltpu.VMEM((CHUNK, D), jnp.float32),
        pltpu.SemaphoreType.DMA,
    ],
    compiler_params=pltpu.CompilerParams(use_tc_tiling_on_sc=False),
)
def _emb_lookup(x_hbm, table_hbm, out_hbm, idx_v, rows_v, sem):
    wid = lax.axis_index("s") * NC + lax.axis_index("c")
    pltpu.sync_copy(x_hbm.at[wid], idx_v)
    base = wid * BPW

    def body(c, carry):
        pltpu.async_copy(table_hbm.at[idx_v.at[c]], rows_v, sem).wait()
        pltpu.sync_copy(rows_v, out_hbm.at[pl.ds(base + c * CHUNK, CHUNK)])
        return carry

    lax.fori_loop(0, NCH, body, 0)


def kernel(x, table):
    xf = x.reshape(NW, NCH, CHUNK)
    out = _emb_lookup(xf, table)
    return out.reshape(x.shape[0], x.shape[1], D)


# speedup vs baseline: 1.8739x; 1.1121x over previous
import functools

import jax
import jax.numpy as jnp
from jax import lax
from jax.experimental import pallas as pl
from jax.experimental.pallas import tpu as pltpu
from jax.experimental.pallas import tpu_sc as plsc

VOCAB = 1000000
D = 64
B = 16384 * 50
NC = 2
NS = 16
NW = NC * NS
BPW = B // NW
CHUNK = 128
NCH = BPW // CHUNK
NBUF = 4
SLAB = NBUF * CHUNK
NR = BPW // SLAB

_mesh = plsc.VectorSubcoreMesh(core_axis_name="c", subcore_axis_name="s")


@functools.partial(
    pl.kernel,
    out_type=jax.ShapeDtypeStruct((B, D), jnp.float32),
    mesh=_mesh,
    scratch_types=[
        pltpu.VMEM((NCH, CHUNK), jnp.int32),
        pltpu.VMEM((2, SLAB, D), jnp.float32),
        pltpu.SemaphoreType.DMA,
        pltpu.SemaphoreType.DMA((2,)),
    ],
    compiler_params=pltpu.CompilerParams(use_tc_tiling_on_sc=False),
)
def _emb_lookup(x_hbm, table_hbm, out_hbm, idx_v, rows_v, gsem, ssem):
    wid = lax.axis_index("s") * NC + lax.axis_index("c")
    pltpu.sync_copy(x_hbm.at[wid], idx_v)
    base = wid * BPW

    def issue_gathers(r, sl):
        for b in range(NBUF):
            pltpu.async_copy(
                table_hbm.at[idx_v.at[r * NBUF + b]],
                rows_v.at[sl, pl.ds(b * CHUNK, CHUNK)],
                gsem,
            )

    def wait_gathers(sl):
        pltpu.make_async_copy(
            table_hbm.at[pl.ds(0, SLAB)], rows_v.at[sl], gsem
        ).wait()

    def issue_scatter(r, sl):
        pltpu.async_copy(
            rows_v.at[sl], out_hbm.at[pl.ds(base + r * SLAB, SLAB)], ssem.at[sl]
        )

    def wait_scatter(r, sl):
        pltpu.make_async_copy(
            rows_v.at[sl], out_hbm.at[pl.ds(base + r * SLAB, SLAB)], ssem.at[sl]
        ).wait()

    issue_gathers(0, 0)
    wait_gathers(0)
    issue_scatter(0, 0)
    issue_gathers(1, 1)

    @pl.loop(0, (NR - 2) // 2)
    def _(i):
        r = 2 * i + 1
        wait_gathers(1)
        issue_scatter(r, 1)
        wait_scatter(r - 1, 0)
        issue_gathers(r + 1, 0)
        wait_gathers(0)
        issue_scatter(r + 1, 0)
        wait_scatter(r, 1)
        issue_gathers(r + 2, 1)

    wait_gathers(1)
    issue_scatter(NR - 1, 1)
    wait_scatter(NR - 2, 0)
    wait_scatter(NR - 1, 1)


def kernel(x, table):
    xf = x.reshape(NW, NCH, CHUNK)
    out = _emb_lookup(xf, table)
    return out.reshape(x.shape[0], x.shape[1], D)
